# baseline (speedup 1.0000x reference)
import functools

import jax
import jax.numpy as jnp
from jax import lax
from jax.experimental import pallas as pl
from jax.experimental.pallas import tpu as pltpu
from jax.experimental.pallas import tpu_sc as plsc

N_LAYERS = 4
N_P = 2048
N_K = 512
E_K = 16384
E_P = 131072

HALF_P = N_P // 2
QTR_P = N_P // 4
HALF_K = N_K // 2
EP_T = E_P // 16
EK_T = E_K // 16

TRASH_AP = QTR_P * N_P
TRASH_AK = HALF_K * N_K
TRASH_CP = HALF_P
TRASH_CK = HALF_K


def _sc_body(knn_src, knn_dst, ppi_src, ppi_dst, zf_h, of_h,
             akt_out, ap_out,
             aP, aK,
             ed, es, kd, ks,
             pidx2d, kidx2d,
             onesf, zf, bb, sem):
    c = lax.axis_index("c")
    s = lax.axis_index("s")

    cp_ed = pltpu.async_copy(ppi_dst.at[pl.ds(s * EP_T, EP_T)], ed, sem)
    cp_es = pltpu.async_copy(ppi_src.at[pl.ds(s * EP_T, EP_T)], es, sem)
    cp_kd = pltpu.async_copy(knn_dst.at[pl.ds(s * EK_T, EK_T)], kd, sem)
    cp_ks = pltpu.async_copy(knn_src.at[pl.ds(s * EK_T, EK_T)], ks, sem)
    pltpu.sync_copy(zf_h, zf)
    pltpu.sync_copy(of_h, onesf)

    pltpu.sync_copy(zf, aK.at[pl.ds(s * 8192, 8192)])

    @pl.when(s == 0)
    def _zero_trash():
        pltpu.sync_copy(zf.at[pl.ds(0, 128)], aK.at[pl.ds(TRASH_AK, 128)])

    cp_ed.wait()
    cp_es.wait()
    cp_kd.wait()
    cp_ks.wait()

    lo_q0 = c * QTR_P
    lo_q1 = (2 + c) * QTR_P
    lo_k = c * HALF_K

    def compact(dref, sref, lo, width, mult, n_edges, idx2d, trash):
        iota = lax.iota(jnp.int32, 16)
        def body(i, offv):
            o = i * 16
            d = dref[pl.ds(o, 16)]
            sv = sref[pl.ds(o, 16)]
            m = (d >= lo) & (d < lo + width)
            mi = m.astype(jnp.int32)
            vals = jnp.where(m, (d - lo) * mult + sv, trash)
            pos = offv + mi
            idx2d[i >> 3, pl.ds((o & 127) * 1, 16)] = vals + pos * 0
            return offv
        offv = lax.fori_loop(0, n_edges // 16, body,
                             jnp.zeros((16,), jnp.int32))
        return 0

    def scatter_rows(idx2d, target, nrows, static_rows):
        def chunk(ci, _):
            base = ci * 8
            for j in range(8):
                pltpu.async_copy(onesf, target.at[idx2d.at[base + j]],
                                 sem, add=True)
            for j in range(8):
                pltpu.make_async_copy(onesf, target.at[idx2d.at[0]],
                                      sem).wait()
            return _
        lax.fori_loop(0, static_rows // 8, chunk, None)

    with jax.named_scope("knn_scan"):
        nk = compact(ks, kd, lo_k, HALF_K, N_K, EK_T, kidx2d, TRASH_AK)

    plsc.subcore_barrier()
    with jax.named_scope("knn_scatter"):
        scatter_rows(kidx2d, aK, nk, 8)

    for rnd, lo_q in enumerate((lo_q0, lo_q1)):
        with jax.named_scope("ppi_scan"):
            np_ = compact(ed, es, lo_q, QTR_P, N_P, EP_T, pidx2d, TRASH_AP)
        with jax.named_scope("ppi_zero"):
            for j in range(8):
                pltpu.sync_copy(zf, aP.at[pl.ds(s * 65536 + j * 8192, 8192)])
            @pl.when(s == 0)
            def _zt():
                pltpu.sync_copy(zf.at[pl.ds(0, 128)],
                                aP.at[pl.ds(TRASH_AP, 128)])
        plsc.subcore_barrier()
        with jax.named_scope("ppi_scatter"):
            scatter_rows(pidx2d, aP, np_, 64)
        plsc.subcore_barrier()
        with jax.named_scope("ppi_copyout"):
            for j in range(8):
                pltpu.sync_copy(aP.at[pl.ds(s * 65536 + j * 8192, 8192)], bb)
                pltpu.sync_copy(bb, ap_out.at[pl.ds(
                    (2 * rnd + c) * (QTR_P * N_P) + s * 65536 + j * 8192,
                    8192)])
        plsc.subcore_barrier()

    with jax.named_scope("knn_copyout"):
        pltpu.sync_copy(aK.at[pl.ds(s * 8192, 8192)], bb)
        pltpu.sync_copy(bb, akt_out.at[pl.ds(c * (HALF_K * N_K) + s * 8192,
                                             8192)])


def _sc_build(knn_src, knn_dst, ppi_src, ppi_dst):
    zf_h = jnp.zeros((8192,), jnp.float32)
    of_h = jnp.ones((128,), jnp.float32)
    mesh = plsc.VectorSubcoreMesh(core_axis_name="c", subcore_axis_name="s")
    f = pl.kernel(
        _sc_body,
        out_type=(
            jax.ShapeDtypeStruct((N_K * N_K,), jnp.float32),
            jax.ShapeDtypeStruct((N_P * N_P,), jnp.float32),
        ),
        mesh=mesh,
        scratch_types=(
            pltpu.VMEM_SHARED((QTR_P * N_P + 128,), jnp.float32),
            pltpu.VMEM_SHARED((HALF_K * N_K + 128,), jnp.float32),
            pltpu.VMEM((EP_T,), jnp.int32),
            pltpu.VMEM((EP_T,), jnp.int32),
            pltpu.VMEM((EK_T,), jnp.int32),
            pltpu.VMEM((EK_T,), jnp.int32),
            pltpu.VMEM((EP_T // 128 + 1, 128), jnp.int32),
            pltpu.VMEM((EK_T // 128 + 1, 128), jnp.int32),
            pltpu.VMEM((128,), jnp.float32),
            pltpu.VMEM((8192,), jnp.float32),
            pltpu.VMEM((8192,), jnp.float32),
            pltpu.SemaphoreType.DMA,
        ),
    )
    return f(knn_src, knn_dst, ppi_src, ppi_dst, zf_h, of_h)


_TILE = 256
_NT = N_P // _TILE


def _nt_dot(a, b):
    return lax.dot_general(a, b, (((1,), (1,)), ((), ())),
                           preferred_element_type=jnp.float32)


def _cols_kernel(e_ref, akt_ref, wl_ref, wr_ref, bl_ref, out_ref,
                 mk_ref, ebf_ref):
    t = pl.program_id(0)

    @pl.when(t == 0)
    def _():
        ebf = e_ref[...].astype(jnp.bfloat16)
        ebf_ref[...] = ebf
        cnt = jnp.sum(akt_ref[...], axis=0, keepdims=True)
        inv = 1.0 / jnp.maximum(cnt, 1.0)
        akt16 = akt_ref[...].astype(jnp.bfloat16)
        mk = jnp.dot(ebf, akt16, preferred_element_type=jnp.float32) * inv
        mk_ref[...] = mk.astype(jnp.bfloat16)

    h = (jnp.dot(wl_ref[0].astype(jnp.bfloat16), mk_ref[...],
                 preferred_element_type=jnp.float32)
         + jnp.dot(wr_ref[0].astype(jnp.bfloat16), ebf_ref[...],
                   preferred_element_type=jnp.float32)
         + bl_ref[0])
    out_ref[...] = jnp.where(h >= 0.0, h, 0.01 * h)


def _cols_stage(e, akt, cols_Wl, cols_Wr, bl3d, i):
    return pl.pallas_call(
        _cols_kernel,
        grid=(_NT,),
        in_specs=[
            pl.BlockSpec((N_P, N_K), lambda t: (0, 0)),
            pl.BlockSpec((N_K, N_K), lambda t: (0, 0)),
            pl.BlockSpec((1, _TILE, N_P), lambda t: (i, t, 0)),
            pl.BlockSpec((1, _TILE, N_P), lambda t: (i, t, 0)),
            pl.BlockSpec((1, _TILE, 1), lambda t: (i, t, 0)),
        ],
        out_specs=pl.BlockSpec((_TILE, N_K), lambda t: (t, 0)),
        out_shape=jax.ShapeDtypeStruct((N_P, N_K), jnp.float32),
        scratch_shapes=[pltpu.VMEM((N_P, N_K), jnp.bfloat16),
                        pltpu.VMEM((N_P, N_K), jnp.bfloat16)],
    )(e, akt, cols_Wl, cols_Wr, bl3d)


def _rows_kernel(ap_ref, e1_ref, wl_ref, wr_ref, rbl_ref, out_ref,
                 e1bf_ref):
    t = pl.program_id(0)

    @pl.when(t == 0)
    def _():
        e1bf_ref[...] = e1_ref[...].astype(jnp.bfloat16)

    ap = ap_ref[...]
    apbf = ap.astype(jnp.bfloat16)
    agg = jnp.dot(apbf, e1bf_ref[...], preferred_element_type=jnp.float32)
    cnt = jnp.sum(ap, axis=1, keepdims=True)
    inv = 1.0 / jnp.maximum(cnt, 1.0)
    mp = (agg * inv).astype(jnp.bfloat16)
    e1t = e1bf_ref[pl.ds(t * _TILE, _TILE), :]
    h = (_nt_dot(mp, wl_ref[0].astype(jnp.bfloat16))
         + _nt_dot(e1t, wr_ref[0].astype(jnp.bfloat16))
         + rbl_ref[0])
    out_ref[...] = jnp.where(h >= 0.0, h, 0.01 * h)


def _rows_stage(e1, ap, rows_Wl, rows_Wr, rbl3d, i):
    return pl.pallas_call(
        _rows_kernel,
        grid=(_NT,),
        in_specs=[
            pl.BlockSpec((_TILE, N_P), lambda t: (t, 0)),
            pl.BlockSpec((N_P, N_K), lambda t: (0, 0)),
            pl.BlockSpec((1, N_K, N_K), lambda t: (i, 0, 0)),
            pl.BlockSpec((1, N_K, N_K), lambda t: (i, 0, 0)),
            pl.BlockSpec((1, 1, N_K), lambda t: (i, 0, 0)),
        ],
        out_specs=pl.BlockSpec((_TILE, N_K), lambda t: (t, 0)),
        out_shape=jax.ShapeDtypeStruct((N_P, N_K), jnp.float32),
        scratch_shapes=[pltpu.VMEM((N_P, N_K), jnp.bfloat16)],
    )(ap, e1, rows_Wl, rows_Wr, rbl3d)


def kernel(x, knn_edge_index, ppi_edge_index, cols_Wl, cols_bl, cols_Wr,
           rows_Wl, rows_bl, rows_Wr):
    akt_flat, ap_flat = _sc_build(
        knn_edge_index[0], knn_edge_index[1],
        ppi_edge_index[0], ppi_edge_index[1])
    akt = akt_flat.reshape(N_K, N_K)
    ap = ap_flat.reshape(N_P, N_P)
    bl3d = cols_bl.reshape(N_LAYERS, N_P, 1)
    rbl3d = rows_bl.reshape(N_LAYERS, 1, N_K)

    e = x
    for i in range(N_LAYERS):
        e = _cols_stage(e, akt, cols_Wl, cols_Wr, bl3d, i)
        e = _rows_stage(e, ap, rows_Wl, rows_Wr, rbl3d, i)
    return e


# speedup vs baseline: 2.0197x; 2.0197x over previous
import functools

import jax
import jax.numpy as jnp
from jax import lax
from jax.experimental import pallas as pl
from jax.experimental.pallas import tpu as pltpu
from jax.experimental.pallas import tpu_sc as plsc

N_LAYERS = 4
N_P = 2048
N_K = 512
E_K = 16384
E_P = 131072

HALF_P = N_P // 2
QTR_P = N_P // 4
HALF_K = N_K // 2
EP_T = E_P // 16
EK_T = E_K // 16

TRASH_AP = QTR_P * N_P
TRASH_AK = HALF_K * N_K
TRASH_CP = HALF_P
TRASH_CK = HALF_K


def _sc_body(knn_src, knn_dst, ppi_src, ppi_dst, zf_h, of_h,
             akt_out, ap_out,
             aP, aK,
             ed, es, kd, ks,
             pidx2d, kidx2d,
             onesf, zf, bb, sem):
    c = lax.axis_index("c")
    s = lax.axis_index("s")

    cp_ed = pltpu.async_copy(ppi_dst.at[pl.ds(s * EP_T, EP_T)], ed, sem)
    cp_es = pltpu.async_copy(ppi_src.at[pl.ds(s * EP_T, EP_T)], es, sem)
    cp_kd = pltpu.async_copy(knn_dst.at[pl.ds(s * EK_T, EK_T)], kd, sem)
    cp_ks = pltpu.async_copy(knn_src.at[pl.ds(s * EK_T, EK_T)], ks, sem)
    pltpu.sync_copy(zf_h, zf)
    pltpu.sync_copy(of_h, onesf)

    pltpu.sync_copy(zf, aK.at[pl.ds(s * 8192, 8192)])

    @pl.when(s == 0)
    def _zero_trash():
        pltpu.sync_copy(zf.at[pl.ds(0, 128)], aK.at[pl.ds(TRASH_AK, 128)])

    cp_ed.wait()
    cp_es.wait()
    cp_kd.wait()
    cp_ks.wait()

    lo_q0 = c * QTR_P
    lo_q1 = (2 + c) * QTR_P
    lo_k = c * HALF_K

    def scatter_rows(idx2d, target, nrows, csz):
        def chunk(ci, _):
            base = ci * csz
            for j in range(csz):
                pltpu.async_copy(onesf, target.at[idx2d.at[base + j]],
                                 sem, add=True)
            for j in range(csz):
                pltpu.make_async_copy(onesf, target.at[idx2d.at[0]],
                                      sem).wait()
            return _
        lax.fori_loop(0, nrows // csz, chunk, None)

    def scan_idx(dref, sref, lo, width, mult, n_rows, idx2d, trash):
        def body(r, _):
            for j in range(8):
                o = r * 128 + j * 16
                d = dref[pl.ds(o, 16)]
                sv = sref[pl.ds(o, 16)]
                m = (d >= lo) & (d < lo + width)
                idx2d[r, pl.ds(j * 16, 16)] = jnp.where(
                    m, (d - lo) * mult + sv, trash + sv)
            return _
        lax.fori_loop(0, n_rows, body, None)

    with jax.named_scope("knn_scan"):
        scan_idx(ks, kd, lo_k, HALF_K, N_K, EK_T // 128, kidx2d, TRASH_AK)

    plsc.subcore_barrier()
    with jax.named_scope("knn_scatter"):
        scatter_rows(kidx2d, aK, EK_T // 128, 8)

    for rnd, lo_q in enumerate((lo_q0, lo_q1)):
        with jax.named_scope("ppi_scan"):
            scan_idx(ed, es, lo_q, QTR_P, N_P, EP_T // 128, pidx2d, TRASH_AP)
        with jax.named_scope("ppi_zero"):
            for j in range(8):
                pltpu.sync_copy(zf, aP.at[pl.ds(s * 65536 + j * 8192, 8192)])
            @pl.when(s == 0)
            def _zt():
                pltpu.sync_copy(zf.at[pl.ds(0, 128)],
                                aP.at[pl.ds(TRASH_AP, 128)])
        plsc.subcore_barrier()
        with jax.named_scope("ppi_scatter"):
            scatter_rows(pidx2d, aP, EP_T // 128, 16)
        plsc.subcore_barrier()
        with jax.named_scope("ppi_copyout"):
            for j in range(8):
                pltpu.sync_copy(aP.at[pl.ds(s * 65536 + j * 8192, 8192)], bb)
                pltpu.sync_copy(bb, ap_out.at[pl.ds(
                    (2 * rnd + c) * (QTR_P * N_P) + s * 65536 + j * 8192,
                    8192)])
        plsc.subcore_barrier()

    with jax.named_scope("knn_copyout"):
        pltpu.sync_copy(aK.at[pl.ds(s * 8192, 8192)], bb)
        pltpu.sync_copy(bb, akt_out.at[pl.ds(c * (HALF_K * N_K) + s * 8192,
                                             8192)])


def _sc_build(knn_src, knn_dst, ppi_src, ppi_dst):
    zf_h = jnp.zeros((8192,), jnp.float32)
    of_h = jnp.ones((128,), jnp.float32)
    mesh = plsc.VectorSubcoreMesh(core_axis_name="c", subcore_axis_name="s")
    f = pl.kernel(
        _sc_body,
        out_type=(
            jax.ShapeDtypeStruct((N_K * N_K,), jnp.float32),
            jax.ShapeDtypeStruct((N_P * N_P,), jnp.float32),
        ),
        mesh=mesh,
        scratch_types=(
            pltpu.VMEM_SHARED((QTR_P * N_P + N_P + 128,), jnp.float32),
            pltpu.VMEM_SHARED((HALF_K * N_K + N_K + 128,), jnp.float32),
            pltpu.VMEM((EP_T,), jnp.int32),
            pltpu.VMEM((EP_T,), jnp.int32),
            pltpu.VMEM((EK_T,), jnp.int32),
            pltpu.VMEM((EK_T,), jnp.int32),
            pltpu.VMEM((EP_T // 128 + 1, 128), jnp.int32),
            pltpu.VMEM((EK_T // 128 + 1, 128), jnp.int32),
            pltpu.VMEM((128,), jnp.float32),
            pltpu.VMEM((8192,), jnp.float32),
            pltpu.VMEM((8192,), jnp.float32),
            pltpu.SemaphoreType.DMA,
        ),
    )
    return f(knn_src, knn_dst, ppi_src, ppi_dst, zf_h, of_h)


_TILE = 256
_NT = N_P // _TILE


def _nt_dot(a, b):
    return lax.dot_general(a, b, (((1,), (1,)), ((), ())),
                           preferred_element_type=jnp.float32)


def _cols_kernel(e_ref, akt_ref, wl_ref, wr_ref, bl_ref, out_ref,
                 mk_ref, ebf_ref):
    t = pl.program_id(0)

    @pl.when(t == 0)
    def _():
        ebf = e_ref[...].astype(jnp.bfloat16)
        ebf_ref[...] = ebf
        cnt = jnp.sum(akt_ref[...], axis=0, keepdims=True)
        inv = 1.0 / jnp.maximum(cnt, 1.0)
        akt16 = akt_ref[...].astype(jnp.bfloat16)
        mk = jnp.dot(ebf, akt16, preferred_element_type=jnp.float32) * inv
        mk_ref[...] = mk.astype(jnp.bfloat16)

    h = (jnp.dot(wl_ref[0].astype(jnp.bfloat16), mk_ref[...],
                 preferred_element_type=jnp.float32)
         + jnp.dot(wr_ref[0].astype(jnp.bfloat16), ebf_ref[...],
                   preferred_element_type=jnp.float32)
         + bl_ref[0])
    out_ref[...] = jnp.where(h >= 0.0, h, 0.01 * h)


def _cols_stage(e, akt, cols_Wl, cols_Wr, bl3d, i):
    return pl.pallas_call(
        _cols_kernel,
        grid=(_NT,),
        in_specs=[
            pl.BlockSpec((N_P, N_K), lambda t: (0, 0)),
            pl.BlockSpec((N_K, N_K), lambda t: (0, 0)),
            pl.BlockSpec((1, _TILE, N_P), lambda t: (i, t, 0)),
            pl.BlockSpec((1, _TILE, N_P), lambda t: (i, t, 0)),
            pl.BlockSpec((1, _TILE, 1), lambda t: (i, t, 0)),
        ],
        out_specs=pl.BlockSpec((_TILE, N_K), lambda t: (t, 0)),
        out_shape=jax.ShapeDtypeStruct((N_P, N_K), jnp.float32),
        scratch_shapes=[pltpu.VMEM((N_P, N_K), jnp.bfloat16),
                        pltpu.VMEM((N_P, N_K), jnp.bfloat16)],
    )(e, akt, cols_Wl, cols_Wr, bl3d)


def _rows_kernel(ap_ref, e1_ref, wl_ref, wr_ref, rbl_ref, out_ref,
                 e1bf_ref):
    t = pl.program_id(0)

    @pl.when(t == 0)
    def _():
        e1bf_ref[...] = e1_ref[...].astype(jnp.bfloat16)

    ap = ap_ref[...]
    apbf = ap.astype(jnp.bfloat16)
    agg = jnp.dot(apbf, e1bf_ref[...], preferred_element_type=jnp.float32)
    cnt = jnp.sum(ap, axis=1, keepdims=True)
    inv = 1.0 / jnp.maximum(cnt, 1.0)
    mp = (agg * inv).astype(jnp.bfloat16)
    e1t = e1bf_ref[pl.ds(t * _TILE, _TILE), :]
    h = (_nt_dot(mp, wl_ref[0].astype(jnp.bfloat16))
         + _nt_dot(e1t, wr_ref[0].astype(jnp.bfloat16))
         + rbl_ref[0])
    out_ref[...] = jnp.where(h >= 0.0, h, 0.01 * h)


def _rows_stage(e1, ap, rows_Wl, rows_Wr, rbl3d, i):
    return pl.pallas_call(
        _rows_kernel,
        grid=(_NT,),
        in_specs=[
            pl.BlockSpec((_TILE, N_P), lambda t: (t, 0)),
            pl.BlockSpec((N_P, N_K), lambda t: (0, 0)),
            pl.BlockSpec((1, N_K, N_K), lambda t: (i, 0, 0)),
            pl.BlockSpec((1, N_K, N_K), lambda t: (i, 0, 0)),
            pl.BlockSpec((1, 1, N_K), lambda t: (i, 0, 0)),
        ],
        out_specs=pl.BlockSpec((_TILE, N_K), lambda t: (t, 0)),
        out_shape=jax.ShapeDtypeStruct((N_P, N_K), jnp.float32),
        scratch_shapes=[pltpu.VMEM((N_P, N_K), jnp.bfloat16)],
    )(ap, e1, rows_Wl, rows_Wr, rbl3d)


def kernel(x, knn_edge_index, ppi_edge_index, cols_Wl, cols_bl, cols_Wr,
           rows_Wl, rows_bl, rows_Wr):
    akt_flat, ap_flat = _sc_build(
        knn_edge_index[0], knn_edge_index[1],
        ppi_edge_index[0], ppi_edge_index[1])
    akt = akt_flat.reshape(N_K, N_K)
    ap = ap_flat.reshape(N_P, N_P)
    bl3d = cols_bl.reshape(N_LAYERS, N_P, 1)
    rbl3d = rows_bl.reshape(N_LAYERS, 1, N_K)

    e = x
    for i in range(N_LAYERS):
        e = _cols_stage(e, akt, cols_Wl, cols_Wr, bl3d, i)
        e = _rows_stage(e, ap, rows_Wl, rows_Wr, rbl3d, i)
    return e


# speedup vs baseline: 2.0203x; 1.0003x over previous
import functools

import jax
import jax.numpy as jnp
from jax import lax
from jax.experimental import pallas as pl
from jax.experimental.pallas import tpu as pltpu
from jax.experimental.pallas import tpu_sc as plsc

N_LAYERS = 4
N_P = 2048
N_K = 512
E_K = 16384
E_P = 131072

HALF_P = N_P // 2
QTR_P = N_P // 4
HALF_K = N_K // 2
EP_T = E_P // 16
EK_T = E_K // 16

TRASH_AP = QTR_P * N_P
TRASH_AK = HALF_K * N_K
TRASH_CP = HALF_P
TRASH_CK = HALF_K


def _sc_body(knn_src, knn_dst, ppi_src, ppi_dst, zf_h, of_h,
             akt_out, ap_out,
             aP, aK,
             ed, es, kd, ks,
             pidx2d, kidx2d,
             onesf, zf, bb, sem):
    c = lax.axis_index("c")
    s = lax.axis_index("s")

    cp_ed = pltpu.async_copy(ppi_dst.at[pl.ds(s * EP_T, EP_T)], ed, sem)
    cp_es = pltpu.async_copy(ppi_src.at[pl.ds(s * EP_T, EP_T)], es, sem)
    cp_kd = pltpu.async_copy(knn_dst.at[pl.ds(s * EK_T, EK_T)], kd, sem)
    cp_ks = pltpu.async_copy(knn_src.at[pl.ds(s * EK_T, EK_T)], ks, sem)
    pltpu.sync_copy(zf_h, zf)
    pltpu.sync_copy(of_h, onesf)

    pltpu.sync_copy(zf, aK.at[pl.ds(s * 8192, 8192)])

    @pl.when(s == 0)
    def _zero_trash():
        pltpu.sync_copy(zf.at[pl.ds(0, 128)], aK.at[pl.ds(TRASH_AK, 128)])

    cp_ed.wait()
    cp_es.wait()
    cp_kd.wait()
    cp_ks.wait()

    lo_q0 = c * QTR_P
    lo_q1 = (2 + c) * QTR_P
    lo_k = c * HALF_K

    def scatter_rows(idx2d, target, nrows, csz):
        def chunk(ci, _):
            base = ci * csz
            for j in range(csz):
                pltpu.async_copy(onesf, target.at[idx2d.at[base + j]],
                                 sem, add=True)
            for j in range(csz):
                pltpu.make_async_copy(onesf, target.at[idx2d.at[0]],
                                      sem).wait()
            return _
        lax.fori_loop(0, nrows // csz, chunk, None)

    def scan_idx(dref, sref, lo, width, mult, n_rows, idx2d, trash):
        def body(r, _):
            for j in range(8):
                o = r * 128 + j * 16
                d = dref[pl.ds(o, 16)]
                sv = sref[pl.ds(o, 16)]
                m = (d >= lo) & (d < lo + width)
                idx2d[r, pl.ds(j * 16, 16)] = jnp.where(
                    m, (d - lo) * mult + sv, trash + sv)
            return _
        lax.fori_loop(0, n_rows, body, None)

    with jax.named_scope("knn_scan"):
        scan_idx(ks, kd, lo_k, HALF_K, N_K, EK_T // 128, kidx2d, TRASH_AK)

    plsc.subcore_barrier()
    with jax.named_scope("knn_scatter"):
        scatter_rows(kidx2d, aK, EK_T // 128, 8)

    for rnd, lo_q in enumerate((lo_q0, lo_q1)):
        with jax.named_scope("ppi_scan"):
            scan_idx(ed, es, lo_q, QTR_P, N_P, EP_T // 128, pidx2d, TRASH_AP)
        with jax.named_scope("ppi_zero"):
            for j in range(8):
                pltpu.sync_copy(zf, aP.at[pl.ds(s * 65536 + j * 8192, 8192)])
            @pl.when(s == 0)
            def _zt():
                pltpu.sync_copy(zf.at[pl.ds(0, 128)],
                                aP.at[pl.ds(TRASH_AP, 128)])
        plsc.subcore_barrier()
        with jax.named_scope("ppi_scatter"):
            scatter_rows(pidx2d, aP, EP_T // 128, 16)
        plsc.subcore_barrier()
        with jax.named_scope("ppi_copyout"):
            for j in range(8):
                pltpu.sync_copy(aP.at[pl.ds(s * 65536 + j * 8192, 8192)], bb)
                pltpu.sync_copy(bb, ap_out.at[pl.ds(
                    (2 * rnd + c) * (QTR_P * N_P) + s * 65536 + j * 8192,
                    8192)])
        plsc.subcore_barrier()

    with jax.named_scope("knn_copyout"):
        pltpu.sync_copy(aK.at[pl.ds(s * 8192, 8192)], bb)
        pltpu.sync_copy(bb, akt_out.at[pl.ds(c * (HALF_K * N_K) + s * 8192,
                                             8192)])


def _sc_build(knn_src, knn_dst, ppi_src, ppi_dst):
    zf_h = jnp.zeros((8192,), jnp.float32)
    of_h = jnp.ones((128,), jnp.float32)
    mesh = plsc.VectorSubcoreMesh(core_axis_name="c", subcore_axis_name="s")
    f = pl.kernel(
        _sc_body,
        out_type=(
            jax.ShapeDtypeStruct((N_K * N_K,), jnp.float32),
            jax.ShapeDtypeStruct((N_P * N_P,), jnp.float32),
        ),
        mesh=mesh,
        scratch_types=(
            pltpu.VMEM_SHARED((QTR_P * N_P + N_P + 128,), jnp.float32),
            pltpu.VMEM_SHARED((HALF_K * N_K + N_K + 128,), jnp.float32),
            pltpu.VMEM((EP_T,), jnp.int32),
            pltpu.VMEM((EP_T,), jnp.int32),
            pltpu.VMEM((EK_T,), jnp.int32),
            pltpu.VMEM((EK_T,), jnp.int32),
            pltpu.VMEM((EP_T // 128 + 1, 128), jnp.int32),
            pltpu.VMEM((EK_T // 128 + 1, 128), jnp.int32),
            pltpu.VMEM((128,), jnp.float32),
            pltpu.VMEM((8192,), jnp.float32),
            pltpu.VMEM((8192,), jnp.float32),
            pltpu.SemaphoreType.DMA,
        ),
    )
    return f(knn_src, knn_dst, ppi_src, ppi_dst, zf_h, of_h)


_TILE = 256
_NT = N_P // _TILE


def _nt_dot(a, b):
    return lax.dot_general(a, b, (((1,), (1,)), ((), ())),
                           preferred_element_type=jnp.float32)


def _cols_kernel(e_ref, akt_ref, wl_ref, wr_ref, bl_ref, out_ref,
                 mk_ref, ebf_ref):
    t = pl.program_id(0)

    @pl.when(t == 0)
    def _():
        ebf = e_ref[...].astype(jnp.bfloat16)
        ebf_ref[...] = ebf
        cnt = jnp.sum(akt_ref[...], axis=0, keepdims=True)
        inv = 1.0 / jnp.maximum(cnt, 1.0)
        akt16 = akt_ref[...].astype(jnp.bfloat16)
        mk = jnp.dot(ebf, akt16, preferred_element_type=jnp.float32) * inv
        mk_ref[...] = mk.astype(jnp.bfloat16)

    h = (jnp.dot(wl_ref[0].astype(jnp.bfloat16), mk_ref[...],
                 preferred_element_type=jnp.float32)
         + jnp.dot(wr_ref[0].astype(jnp.bfloat16), ebf_ref[...],
                   preferred_element_type=jnp.float32)
         + bl_ref[0])
    out_ref[...] = jnp.where(h >= 0.0, h, 0.01 * h)


def _cols_stage(e, akt, cols_Wl, cols_Wr, bl3d, i):
    return pl.pallas_call(
        _cols_kernel,
        grid=(_NT,),
        in_specs=[
            pl.BlockSpec((N_P, N_K), lambda t: (0, 0)),
            pl.BlockSpec((N_K, N_K), lambda t: (0, 0)),
            pl.BlockSpec((1, _TILE, N_P), lambda t: (i, t, 0)),
            pl.BlockSpec((1, _TILE, N_P), lambda t: (i, t, 0)),
            pl.BlockSpec((1, _TILE, 1), lambda t: (i, t, 0)),
        ],
        out_specs=pl.BlockSpec((_TILE, N_K), lambda t: (t, 0)),
        out_shape=jax.ShapeDtypeStruct((N_P, N_K), jnp.float32),
        scratch_shapes=[pltpu.VMEM((N_P, N_K), jnp.bfloat16),
                        pltpu.VMEM((N_P, N_K), jnp.bfloat16)],
    )(e, akt, cols_Wl, cols_Wr, bl3d)


def _rows_kernel(ap_ref, e1_ref, wl_ref, wr_ref, rbl_ref, out_ref,
                 e1bf_ref):
    t = pl.program_id(0)

    @pl.when(t == 0)
    def _():
        e1bf_ref[...] = e1_ref[...].astype(jnp.bfloat16)

    apbf = ap_ref[...]
    agg = jnp.dot(apbf, e1bf_ref[...], preferred_element_type=jnp.float32)
    cnt = jnp.sum(apbf.astype(jnp.float32), axis=1, keepdims=True)
    inv = 1.0 / jnp.maximum(cnt, 1.0)
    mp = (agg * inv).astype(jnp.bfloat16)
    e1t = e1bf_ref[pl.ds(t * _TILE, _TILE), :]
    h = (_nt_dot(mp, wl_ref[0].astype(jnp.bfloat16))
         + _nt_dot(e1t, wr_ref[0].astype(jnp.bfloat16))
         + rbl_ref[0])
    out_ref[...] = jnp.where(h >= 0.0, h, 0.01 * h)


def _rows_stage(e1, ap, rows_Wl, rows_Wr, rbl3d, i):
    return pl.pallas_call(
        _rows_kernel,
        grid=(_NT,),
        in_specs=[
            pl.BlockSpec((_TILE, N_P), lambda t: (t, 0)),
            pl.BlockSpec((N_P, N_K), lambda t: (0, 0)),
            pl.BlockSpec((1, N_K, N_K), lambda t: (i, 0, 0)),
            pl.BlockSpec((1, N_K, N_K), lambda t: (i, 0, 0)),
            pl.BlockSpec((1, 1, N_K), lambda t: (i, 0, 0)),
        ],
        out_specs=pl.BlockSpec((_TILE, N_K), lambda t: (t, 0)),
        out_shape=jax.ShapeDtypeStruct((N_P, N_K), jnp.float32),
        scratch_shapes=[pltpu.VMEM((N_P, N_K), jnp.bfloat16)],
    )(ap, e1, rows_Wl, rows_Wr, rbl3d)


def kernel(x, knn_edge_index, ppi_edge_index, cols_Wl, cols_bl, cols_Wr,
           rows_Wl, rows_bl, rows_Wr):
    akt_flat, ap_flat = _sc_build(
        knn_edge_index[0], knn_edge_index[1],
        ppi_edge_index[0], ppi_edge_index[1])
    akt = akt_flat.reshape(N_K, N_K)
    ap = ap_flat.reshape(N_P, N_P).astype(jnp.bfloat16)
    bl3d = cols_bl.reshape(N_LAYERS, N_P, 1)
    rbl3d = rows_bl.reshape(N_LAYERS, 1, N_K)

    e = x
    for i in range(N_LAYERS):
        e = _cols_stage(e, akt, cols_Wl, cols_Wr, bl3d, i)
        e = _rows_stage(e, ap, rows_Wl, rows_Wr, rbl3d, i)
    return e


# speedup vs baseline: 2.1372x; 1.0578x over previous
import functools

import jax
import jax.numpy as jnp
from jax import lax
from jax.experimental import pallas as pl
from jax.experimental.pallas import tpu as pltpu
from jax.experimental.pallas import tpu_sc as plsc

N_LAYERS = 4
N_P = 2048
N_K = 512
E_K = 16384
E_P = 131072

HALF_P = N_P // 2
QTR_P = N_P // 4
HALF_K = N_K // 2
EP_T = E_P // 16
EK_T = E_K // 16

TRASH_AP = QTR_P * N_P
TRASH_AK = HALF_K * N_K
TRASH_CP = HALF_P
TRASH_CK = HALF_K


def _sc_body(knn_src, knn_dst, ppi_src, ppi_dst, zf_h, of_h,
             akt_out, ap_out,
             aP, aK,
             ed, es, kd, ks,
             pidx2d, kidx2d,
             onesf, zf, bb, bb2, sem):
    c = lax.axis_index("c")
    s = lax.axis_index("s")

    cp_ed = pltpu.async_copy(ppi_dst.at[pl.ds(s * EP_T, EP_T)], ed, sem)
    cp_es = pltpu.async_copy(ppi_src.at[pl.ds(s * EP_T, EP_T)], es, sem)
    cp_kd = pltpu.async_copy(knn_dst.at[pl.ds(s * EK_T, EK_T)], kd, sem)
    cp_ks = pltpu.async_copy(knn_src.at[pl.ds(s * EK_T, EK_T)], ks, sem)
    pltpu.sync_copy(zf_h, zf)
    pltpu.sync_copy(of_h, onesf)

    pltpu.sync_copy(zf, aK.at[pl.ds(s * 8192, 8192)])

    @pl.when(s == 0)
    def _zero_trash():
        pltpu.sync_copy(zf.at[pl.ds(0, 128)], aK.at[pl.ds(TRASH_AK, 128)])

    cp_ed.wait()
    cp_es.wait()
    cp_kd.wait()
    cp_ks.wait()

    lo_q0 = c * QTR_P
    lo_q1 = (2 + c) * QTR_P
    lo_k = c * HALF_K

    def scatter_rows(idx2d, target, nrows, csz):
        def chunk(ci, _):
            base = ci * csz
            for j in range(csz):
                pltpu.async_copy(onesf, target.at[idx2d.at[base + j]],
                                 sem, add=True)
            for j in range(csz):
                pltpu.make_async_copy(onesf, target.at[idx2d.at[0]],
                                      sem).wait()
            return _
        lax.fori_loop(0, nrows // csz, chunk, None)

    def scan_idx(dref, sref, lo, width, mult, n_rows, idx2d, trash):
        def body(r, _):
            for j in range(8):
                o = r * 128 + j * 16
                d = dref[pl.ds(o, 16)]
                sv = sref[pl.ds(o, 16)]
                m = (d >= lo) & (d < lo + width)
                idx2d[r, pl.ds(j * 16, 16)] = jnp.where(
                    m, (d - lo) * mult + sv, trash + sv)
            return _
        lax.fori_loop(0, n_rows, body, None)

    with jax.named_scope("knn_scan"):
        scan_idx(ks, kd, lo_k, HALF_K, N_K, EK_T // 128, kidx2d, TRASH_AK)

    plsc.subcore_barrier()
    with jax.named_scope("knn_scatter"):
        scatter_rows(kidx2d, aK, EK_T // 128, 8)

    for rnd, lo_q in enumerate((lo_q0, lo_q1)):
        with jax.named_scope("ppi_scan"):
            scan_idx(ed, es, lo_q, QTR_P, N_P, EP_T // 128, pidx2d, TRASH_AP)
        with jax.named_scope("ppi_zero"):
            for j in range(8):
                pltpu.sync_copy(zf, aP.at[pl.ds(s * 65536 + j * 8192, 8192)])
            @pl.when(s == 0)
            def _zt():
                pltpu.sync_copy(zf.at[pl.ds(0, 128)],
                                aP.at[pl.ds(TRASH_AP, 128)])
        plsc.subcore_barrier()
        with jax.named_scope("ppi_scatter"):
            scatter_rows(pidx2d, aP, EP_T // 128, 16)
        plsc.subcore_barrier()
        with jax.named_scope("ppi_copyout"):
            hs = [None, None]
            for j in range(8):
                b = (bb, bb2)[j % 2]
                if hs[j % 2] is not None:
                    hs[j % 2].wait()
                pltpu.sync_copy(aP.at[pl.ds(s * 65536 + j * 8192, 8192)], b)
                hs[j % 2] = pltpu.async_copy(b, ap_out.at[pl.ds(
                    (2 * rnd + c) * (QTR_P * N_P) + s * 65536 + j * 8192,
                    8192)], sem)
            hs[0].wait()
            hs[1].wait()
        plsc.subcore_barrier()

    with jax.named_scope("knn_copyout"):
        pltpu.sync_copy(aK.at[pl.ds(s * 8192, 8192)], bb)
        pltpu.sync_copy(bb, akt_out.at[pl.ds(c * (HALF_K * N_K) + s * 8192,
                                             8192)])


def _sc_build(knn_src, knn_dst, ppi_src, ppi_dst):
    zf_h = jnp.zeros((8192,), jnp.float32)
    of_h = jnp.ones((128,), jnp.float32)
    mesh = plsc.VectorSubcoreMesh(core_axis_name="c", subcore_axis_name="s")
    f = pl.kernel(
        _sc_body,
        out_type=(
            jax.ShapeDtypeStruct((N_K * N_K,), jnp.float32),
            jax.ShapeDtypeStruct((N_P * N_P,), jnp.float32),
        ),
        mesh=mesh,
        scratch_types=(
            pltpu.VMEM_SHARED((QTR_P * N_P + N_P + 128,), jnp.float32),
            pltpu.VMEM_SHARED((HALF_K * N_K + N_K + 128,), jnp.float32),
            pltpu.VMEM((EP_T,), jnp.int32),
            pltpu.VMEM((EP_T,), jnp.int32),
            pltpu.VMEM((EK_T,), jnp.int32),
            pltpu.VMEM((EK_T,), jnp.int32),
            pltpu.VMEM((EP_T // 128 + 1, 128), jnp.int32),
            pltpu.VMEM((EK_T // 128 + 1, 128), jnp.int32),
            pltpu.VMEM((128,), jnp.float32),
            pltpu.VMEM((8192,), jnp.float32),
            pltpu.VMEM((8192,), jnp.float32),
            pltpu.VMEM((8192,), jnp.float32),
            pltpu.SemaphoreType.DMA,
        ),
    )
    return f(knn_src, knn_dst, ppi_src, ppi_dst, zf_h, of_h)


_TILE = 512
_NT = N_P // _TILE


def _nt_dot(a, b):
    return lax.dot_general(a, b, (((1,), (1,)), ((), ())),
                           preferred_element_type=jnp.float32)


def _cols_kernel(e_ref, akt_ref, wl_ref, wr_ref, bl_ref, out_ref,
                 mk_ref, ebf_ref):
    t = pl.program_id(0)

    @pl.when(t == 0)
    def _():
        ebf = e_ref[...].astype(jnp.bfloat16)
        ebf_ref[...] = ebf
        cnt = jnp.sum(akt_ref[...], axis=0, keepdims=True)
        inv = 1.0 / jnp.maximum(cnt, 1.0)
        akt16 = akt_ref[...].astype(jnp.bfloat16)
        mk = jnp.dot(ebf, akt16, preferred_element_type=jnp.float32) * inv
        mk_ref[...] = mk.astype(jnp.bfloat16)

    h = (jnp.dot(wl_ref[0].astype(jnp.bfloat16), mk_ref[...],
                 preferred_element_type=jnp.float32)
         + jnp.dot(wr_ref[0].astype(jnp.bfloat16), ebf_ref[...],
                   preferred_element_type=jnp.float32)
         + bl_ref[0])
    out_ref[...] = jnp.where(h >= 0.0, h, 0.01 * h)


def _cols_stage(e, akt, cols_Wl, cols_Wr, bl3d, i):
    return pl.pallas_call(
        _cols_kernel,
        grid=(_NT,),
        in_specs=[
            pl.BlockSpec((N_P, N_K), lambda t: (0, 0)),
            pl.BlockSpec((N_K, N_K), lambda t: (0, 0)),
            pl.BlockSpec((1, _TILE, N_P), lambda t: (i, t, 0)),
            pl.BlockSpec((1, _TILE, N_P), lambda t: (i, t, 0)),
            pl.BlockSpec((1, _TILE, 1), lambda t: (i, t, 0)),
        ],
        out_specs=pl.BlockSpec((_TILE, N_K), lambda t: (t, 0)),
        out_shape=jax.ShapeDtypeStruct((N_P, N_K), jnp.float32),
        scratch_shapes=[pltpu.VMEM((N_P, N_K), jnp.bfloat16),
                        pltpu.VMEM((N_P, N_K), jnp.bfloat16)],
    )(e, akt, cols_Wl, cols_Wr, bl3d)


def _rows_kernel(ap_ref, e1_ref, wl_ref, wr_ref, rbl_ref, out_ref,
                 e1bf_ref):
    t = pl.program_id(0)

    @pl.when(t == 0)
    def _():
        e1bf_ref[...] = e1_ref[...].astype(jnp.bfloat16)

    apbf = ap_ref[...]
    agg = jnp.dot(apbf, e1bf_ref[...], preferred_element_type=jnp.float32)
    cnt = jnp.sum(apbf.astype(jnp.float32), axis=1, keepdims=True)
    inv = 1.0 / jnp.maximum(cnt, 1.0)
    mp = (agg * inv).astype(jnp.bfloat16)
    e1t = e1bf_ref[pl.ds(t * _TILE, _TILE), :]
    h = (_nt_dot(mp, wl_ref[0].astype(jnp.bfloat16))
         + _nt_dot(e1t, wr_ref[0].astype(jnp.bfloat16))
         + rbl_ref[0])
    out_ref[...] = jnp.where(h >= 0.0, h, 0.01 * h)


def _rows_stage(e1, ap, rows_Wl, rows_Wr, rbl3d, i):
    return pl.pallas_call(
        _rows_kernel,
        grid=(_NT,),
        in_specs=[
            pl.BlockSpec((_TILE, N_P), lambda t: (t, 0)),
            pl.BlockSpec((N_P, N_K), lambda t: (0, 0)),
            pl.BlockSpec((1, N_K, N_K), lambda t: (i, 0, 0)),
            pl.BlockSpec((1, N_K, N_K), lambda t: (i, 0, 0)),
            pl.BlockSpec((1, 1, N_K), lambda t: (i, 0, 0)),
        ],
        out_specs=pl.BlockSpec((_TILE, N_K), lambda t: (t, 0)),
        out_shape=jax.ShapeDtypeStruct((N_P, N_K), jnp.float32),
        scratch_shapes=[pltpu.VMEM((N_P, N_K), jnp.bfloat16)],
    )(ap, e1, rows_Wl, rows_Wr, rbl3d)


def kernel(x, knn_edge_index, ppi_edge_index, cols_Wl, cols_bl, cols_Wr,
           rows_Wl, rows_bl, rows_Wr):
    akt_flat, ap_flat = _sc_build(
        knn_edge_index[0], knn_edge_index[1],
        ppi_edge_index[0], ppi_edge_index[1])
    akt = akt_flat.reshape(N_K, N_K)
    ap = ap_flat.reshape(N_P, N_P).astype(jnp.bfloat16)
    bl3d = cols_bl.reshape(N_LAYERS, N_P, 1)
    rbl3d = rows_bl.reshape(N_LAYERS, 1, N_K)

    e = x
    for i in range(N_LAYERS):
        e = _cols_stage(e, akt, cols_Wl, cols_Wr, bl3d, i)
        e = _rows_stage(e, ap, rows_Wl, rows_Wr, rbl3d, i)
    return e
